# Initial kernel scaffold; baseline (speedup 1.0000x reference)
import jax
import jax.numpy as jnp
from jax.experimental import pallas as pl


def kernel(x, adj, W1, b1, W3, b3, W4, b4, W2, b2):
    raise NotImplementedError("write your pallas kernel here")


# speedup vs baseline: 9.7472x; 9.7472x over previous
import functools

import jax
import jax.numpy as jnp
from jax import lax
from jax.experimental import pallas as pl
from jax.experimental.pallas import tpu as pltpu
from jax.experimental.pallas import tpu_sc as plsc

NC = 2
NS = 16
LANES = 16
K = 80
RB = 400


def _sc_mesh():
    return plsc.VectorSubcoreMesh(core_axis_name="c", subcore_axis_name="s")


def _fill_rows(buf, rows, ncols, value):
    vec = jnp.full((LANES,), value, jnp.float32)

    def body(r, _):
        for c in range(ncols // LANES):
            buf[r, pl.ds(c * LANES, LANES)] = vec
        return 0

    lax.fori_loop(0, rows, body, 0)


def _row_part(N):
    assert N % 8 == 0
    rows_main = (N // (NS * 8)) * 8
    tail_groups = (N - NS * rows_main) // 8
    assert tail_groups <= NS
    return rows_main, tail_groups


def _zero_acc(acc, zbuf, sid, N):
    rows_main, tail_groups = _row_part(N)
    r0 = sid * rows_main
    nfull = rows_main // K
    for j in range(nfull):
        pltpu.sync_copy(zbuf, acc.at[pl.ds(r0 + j * K, K)])
    rem = rows_main - nfull * K
    if rem:
        pltpu.sync_copy(zbuf.at[pl.ds(0, rem)], acc.at[pl.ds(r0 + nfull * K, rem)])
    if tail_groups:
        @pl.when(sid < tail_groups)
        def _():
            pltpu.sync_copy(zbuf.at[pl.ds(0, 8)],
                            acc.at[pl.ds(NS * rows_main + sid * 8, 8)])


def _copy_out(acc, out_hbm, cid, sid, N):
    rows_main, tail_groups = _row_part(N)
    r0 = sid * rows_main
    pltpu.sync_copy(acc.at[pl.ds(r0, rows_main)],
                    out_hbm.at[cid, pl.ds(r0, rows_main)])
    if tail_groups:
        @pl.when(sid < tail_groups)
        def _():
            t0 = NS * rows_main + sid * 8
            pltpu.sync_copy(acc.at[pl.ds(t0, 8)], out_hbm.at[cid, pl.ds(t0, 8)])


def _make_count(N, E):
    D = 128
    e_pt = E // (NC * NS)
    assert e_pt % K == 0
    n_chunks = e_pt // K

    @functools.partial(
        pl.kernel,
        mesh=_sc_mesh(),
        out_type=jax.ShapeDtypeStruct((NC, N, D), jnp.float32),
        scratch_types=[
            pltpu.VMEM((K,), jnp.int32),
            pltpu.VMEM((K, D), jnp.float32),
            pltpu.VMEM((K, D), jnp.float32),
            pltpu.VMEM_SHARED((N, D), jnp.float32),
        ],
    )
    def count(col_hbm, out_hbm, cidx, ones_buf, zbuf, acc):
        cid = lax.axis_index("c")
        sid = lax.axis_index("s")
        _fill_rows(ones_buf, K, D, 1.0)
        _fill_rows(zbuf, K, D, 0.0)
        _zero_acc(acc, zbuf, sid, N)
        plsc.subcore_barrier()
        base = (cid * NS + sid) * e_pt

        def body(i, _):
            pltpu.sync_copy(col_hbm.at[pl.ds(base + i * K, K)], cidx)
            pltpu.sync_copy(ones_buf, acc.at[cidx], add=True)
            return 0

        lax.fori_loop(0, n_chunks, body, 0)
        plsc.subcore_barrier()
        _copy_out(acc, out_hbm, cid, sid, N)

    return count


def _make_prop(N, E, D):
    e_pt = E // (NC * NS)
    assert e_pt % K == 0 and D % LANES == 0
    n_chunks = e_pt // K

    @functools.partial(
        pl.kernel,
        mesh=_sc_mesh(),
        out_type=jax.ShapeDtypeStruct((NC, N, D), jnp.float32),
        scratch_types=[
            pltpu.VMEM((K,), jnp.int32),
            pltpu.VMEM((K,), jnp.int32),
            pltpu.VMEM((K, D), jnp.float32),
            pltpu.VMEM_SHARED((N, D), jnp.float32),
            pltpu.SemaphoreType.DMA,
        ],
    )
    def prop(xs_hbm, row_hbm, col_hbm, out_hbm, ridx, cidx, buf, acc, sem):
        cid = lax.axis_index("c")
        sid = lax.axis_index("s")
        _fill_rows(buf, K, D, 0.0)
        _zero_acc(acc, buf, sid, N)
        plsc.subcore_barrier()
        base = (cid * NS + sid) * e_pt

        def body(i, _):
            off = base + i * K
            pltpu.sync_copy(row_hbm.at[pl.ds(off, K)], ridx)
            pltpu.sync_copy(col_hbm.at[pl.ds(off, K)], cidx)
            pltpu.async_copy(xs_hbm.at[ridx], buf, sem).wait()
            pltpu.sync_copy(buf, acc.at[cidx], add=True)
            return 0

        lax.fori_loop(0, n_chunks, body, 0)
        plsc.subcore_barrier()
        _copy_out(acc, out_hbm, cid, sid, N)

    return prop


def _row_specs(N, shapes):
    specs = []
    for shp in shapes:
        if len(shp) == 3:
            specs.append(pl.BlockSpec((NC, RB, shp[2]), lambda i: (0, i, 0)))
        elif shp[0] == N:
            specs.append(pl.BlockSpec((RB, shp[1]), lambda i: (i, 0)))
        else:
            specs.append(pl.BlockSpec(shp, lambda i: (0,) * len(shp)))
    return specs


def _prep(cnt, x, W1):
    N, F = x.shape
    H = W1.shape[1]

    def body(cnt_ref, x_ref, w_ref, dinvt_ref, xs_ref):
        c = cnt_ref[0, :, 0:1] + cnt_ref[1, :, 0:1]
        dinv = lax.rsqrt(c + 1.0)
        dinvt_ref[...] = jnp.broadcast_to(dinv, (RB, 16))
        xs_ref[...] = dinv * jnp.dot(x_ref[...], w_ref[...],
                                     preferred_element_type=jnp.float32)

    return pl.pallas_call(
        body,
        grid=(N // RB,),
        in_specs=_row_specs(N, [cnt.shape, x.shape, W1.shape]),
        out_specs=_row_specs(N, [(N, 16), (N, H)]),
        out_shape=(jax.ShapeDtypeStruct((N, 16), jnp.float32),
                   jax.ShapeDtypeStruct((N, H), jnp.float32)),
    )(cnt, x, W1)


def _weights(W4, W2, b4):
    F, H2 = W4.shape
    C = W2.shape[1]

    def body(w4_ref, w2_ref, b4_ref, w42_ref, v_ref):
        zw = jnp.zeros((F, F - C), jnp.float32)
        zv = jnp.zeros((1, F - C), jnp.float32)
        w42_ref[...] = jnp.concatenate(
            [jnp.dot(w4_ref[...], w2_ref[...],
                     preferred_element_type=jnp.float32), zw], axis=1)
        v_ref[...] = jnp.concatenate(
            [jnp.dot(b4_ref[...], w2_ref[...],
                     preferred_element_type=jnp.float32), zv], axis=1)

    return pl.pallas_call(
        body,
        out_shape=(jax.ShapeDtypeStruct((F, F), jnp.float32),
                   jax.ShapeDtypeStruct((1, F), jnp.float32)),
    )(W4, W2, b4.reshape(1, -1))


def _layer(parts, xs, dinvt, b, W):
    N, D = xs.shape
    Dout = W.shape[1]

    def body(p_ref, xs_ref, dt_ref, b_ref, w_ref, o_ref):
        dinv = dt_ref[:, 0:1]
        h = dinv * (p_ref[0] + p_ref[1] + xs_ref[...]) + b_ref[...]
        h = jnp.tanh(h)
        o_ref[...] = dinv * jnp.dot(h, w_ref[...],
                                    preferred_element_type=jnp.float32)

    return pl.pallas_call(
        body,
        grid=(N // RB,),
        in_specs=_row_specs(N, [parts.shape, xs.shape, dinvt.shape,
                                (1, D), W.shape]),
        out_specs=_row_specs(N, [(N, Dout)])[0],
        out_shape=jax.ShapeDtypeStruct((N, Dout), jnp.float32),
    )(parts, xs, dinvt, b.reshape(1, -1), W)


def _rescale(parts, xs, dinvt, v):
    N, D = xs.shape

    def body(p_ref, xs_ref, dt_ref, v_ref, o_ref):
        dinv = dt_ref[:, 0:1]
        o_ref[...] = dinv * (dinv * (p_ref[0] + p_ref[1] + xs_ref[...])
                             + v_ref[...])

    return pl.pallas_call(
        body,
        grid=(N // RB,),
        in_specs=_row_specs(N, [parts.shape, xs.shape, dinvt.shape, (1, D)]),
        out_specs=_row_specs(N, [(N, D)])[0],
        out_shape=jax.ShapeDtypeStruct((N, D), jnp.float32),
    )(parts, xs, dinvt, v)


def _final(parts, xs, dinvt, b2, C):
    N, D = xs.shape

    def body(p_ref, xs_ref, dt_ref, b2_ref, o_ref):
        dinv = dt_ref[:, 0:1]
        u = dinv * (p_ref[0] + p_ref[1] + xs_ref[...])
        o_ref[...] = u[:, :C] + b2_ref[...]

    return pl.pallas_call(
        body,
        grid=(N // RB,),
        in_specs=_row_specs(N, [parts.shape, xs.shape, dinvt.shape, (1, C)]),
        out_specs=_row_specs(N, [(N, C)])[0],
        out_shape=jax.ShapeDtypeStruct((N, C), jnp.float32),
    )(parts, xs, dinvt, b2.reshape(1, -1))


def kernel(x, adj, W1, b1, W3, b3, W4, b4, W2, b2):
    N, F = x.shape
    E = adj.shape[1]
    H = W1.shape[1]
    C = W2.shape[1]
    row = adj[0].astype(jnp.int32)
    col = adj[1].astype(jnp.int32)

    prop_h = _make_prop(N, E, H)

    cnt = _make_count(N, E)(col)
    dinvt, xs = _prep(cnt, x, W1)
    W42, v = _weights(W4, W2, b4)

    p = prop_h(xs, row, col)
    xs = _layer(p, xs, dinvt, b1, W3)
    p = prop_h(xs, row, col)
    xs = _layer(p, xs, dinvt, b3, W3)
    p = prop_h(xs, row, col)
    xs = _layer(p, xs, dinvt, b3, W42)
    p = prop_h(xs, row, col)
    xs = _rescale(p, xs, dinvt, v)
    p = prop_h(xs, row, col)
    return _final(p, xs, dinvt, b2, C)


# speedup vs baseline: 18.5459x; 1.9027x over previous
import functools

import jax
import jax.numpy as jnp
from jax import lax
from jax.experimental import pallas as pl
from jax.experimental.pallas import tpu as pltpu
from jax.experimental.pallas import tpu_sc as plsc

NC = 2
NS = 16
LANES = 16
K = 80
RB = 400


def _sc_mesh():
    return plsc.VectorSubcoreMesh(core_axis_name="c", subcore_axis_name="s")


def _fill_rows(buf, rows, ncols, value):
    vec = jnp.full((LANES,), value, jnp.float32)

    def body(r, _):
        for c in range(ncols // LANES):
            buf[r, pl.ds(c * LANES, LANES)] = vec
        return 0

    lax.fori_loop(0, rows, body, 0)


def _row_part(N):
    assert N % 8 == 0
    rows_main = (N // (NS * 8)) * 8
    tail_groups = (N - NS * rows_main) // 8
    assert tail_groups <= NS
    return rows_main, tail_groups


def _zero_acc(acc, zbuf, sid, N):
    rows_main, tail_groups = _row_part(N)
    r0 = sid * rows_main
    nfull = rows_main // K
    for j in range(nfull):
        pltpu.sync_copy(zbuf, acc.at[pl.ds(r0 + j * K, K)])
    rem = rows_main - nfull * K
    if rem:
        pltpu.sync_copy(zbuf.at[pl.ds(0, rem)], acc.at[pl.ds(r0 + nfull * K, rem)])
    if tail_groups:
        @pl.when(sid < tail_groups)
        def _():
            pltpu.sync_copy(zbuf.at[pl.ds(0, 8)],
                            acc.at[pl.ds(NS * rows_main + sid * 8, 8)])


def _copy_out(acc, out_hbm, cid, sid, N):
    rows_main, tail_groups = _row_part(N)
    r0 = sid * rows_main
    pltpu.sync_copy(acc.at[pl.ds(r0, rows_main)],
                    out_hbm.at[cid, pl.ds(r0, rows_main)])
    if tail_groups:
        @pl.when(sid < tail_groups)
        def _():
            t0 = NS * rows_main + sid * 8
            pltpu.sync_copy(acc.at[pl.ds(t0, 8)], out_hbm.at[cid, pl.ds(t0, 8)])


def _make_count(N, E):
    D = 16
    e_pt = E // (NC * NS)
    assert e_pt % K == 0
    n_chunks = e_pt // K
    assert n_chunks % 2 == 1 and n_chunks >= 3

    @functools.partial(
        pl.kernel,
        mesh=_sc_mesh(),
        out_type=jax.ShapeDtypeStruct((NC, N, D), jnp.float32),
        scratch_types=[
            pltpu.VMEM((n_chunks, 1, K), jnp.int32),
            pltpu.VMEM((K, D), jnp.float32),
            pltpu.VMEM((K, D), jnp.float32),
            pltpu.VMEM_SHARED((N, D), jnp.float32),
            pltpu.SemaphoreType.DMA,
            pltpu.SemaphoreType.DMA,
        ],
    )
    def count(col_hbm, out_hbm, cbuf, ones_buf, zbuf, acc, s0, s1):
        cid = lax.axis_index("c")
        sid = lax.axis_index("s")
        gid = cid * NS + sid
        _fill_rows(ones_buf, K, D, 1.0)
        _fill_rows(zbuf, K, D, 0.0)
        _zero_acc(acc, zbuf, sid, N)
        pltpu.sync_copy(col_hbm.at[gid], cbuf)
        plsc.subcore_barrier()

        def start(i, sem):
            pltpu.async_copy(ones_buf, acc.at[cbuf.at[i, 0]], sem, add=True)

        def wait(sem):
            pltpu.make_async_copy(ones_buf, acc.at[cbuf.at[0, 0]], sem).wait()

        start(0, s0)
        start(1, s1)

        def body(j, _):
            c = 2 * j
            wait(s0)
            start(c, s0)
            wait(s1)
            start(c + 1, s1)
            return 0

        lax.fori_loop(1, (n_chunks - 1) // 2, body, 0)
        wait(s0)
        start(n_chunks - 1, s0)
        wait(s1)
        wait(s0)
        plsc.subcore_barrier()
        _copy_out(acc, out_hbm, cid, sid, N)

    return count


def _make_prop(N, E, D):
    e_pt = E // (NC * NS)
    assert e_pt % K == 0 and D % LANES == 0
    n_chunks = e_pt // K
    assert n_chunks % 2 == 1 and n_chunks >= 5

    @functools.partial(
        pl.kernel,
        mesh=_sc_mesh(),
        out_type=jax.ShapeDtypeStruct((NC, N, D), jnp.float32),
        scratch_types=[
            pltpu.VMEM((2, 1, K), jnp.int32),
            pltpu.VMEM((2, 1, K), jnp.int32),
            pltpu.VMEM((2, K, D), jnp.float32),
            pltpu.VMEM_SHARED((N, D), jnp.float32),
            pltpu.SemaphoreType.DMA,
            pltpu.SemaphoreType.DMA,
            pltpu.SemaphoreType.DMA,
            pltpu.SemaphoreType.DMA,
        ],
    )
    def prop(xs_hbm, row_hbm, col_hbm, out_hbm, rbuf, cbuf, buf, acc,
             g0, g1, i0, i1):
        cid = lax.axis_index("c")
        sid = lax.axis_index("s")
        gid = cid * NS + sid
        gsem = (g0, g1)
        isem = (i0, i1)
        _fill_rows(buf.at[0], K, D, 0.0)
        _zero_acc(acc, buf.at[0], sid, N)

        def load_idx(s, c, sync=False):
            if sync:
                pltpu.sync_copy(row_hbm.at[gid, c], rbuf.at[s])
                pltpu.sync_copy(col_hbm.at[gid, c], cbuf.at[s])
            else:
                pltpu.async_copy(row_hbm.at[gid, c], rbuf.at[s], isem[s])
                pltpu.async_copy(col_hbm.at[gid, c], cbuf.at[s], isem[s])

        def wait_idx(s):
            pltpu.make_async_copy(row_hbm.at[gid, 0], rbuf.at[s], isem[s]).wait()
            pltpu.make_async_copy(col_hbm.at[gid, 0], cbuf.at[s], isem[s]).wait()

        def start_g(s, c):
            pltpu.async_copy(xs_hbm.at[rbuf.at[s, 0]], buf.at[s], gsem[s])

        def wait_g(s):
            pltpu.make_async_copy(xs_hbm.at[rbuf.at[0, 0]], buf.at[s], gsem[s]).wait()

        def scat(s, c):
            pltpu.sync_copy(buf.at[s], acc.at[cbuf.at[s, 0]], add=True)

        plsc.subcore_barrier()
        load_idx(0, 0, sync=True)
        start_g(0, 0)
        load_idx(1, 1)

        def half(s, c, prefetch, start_next):
            if start_next:
                wait_idx(1 - s)
            wait_g(s)
            if start_next:
                start_g(1 - s, c + 1)
            scat(s, c)
            if prefetch:
                load_idx(s, c + 2)

        def body(j, _):
            c = 2 * j
            half(0, c, True, True)
            half(1, c + 1, True, True)
            return 0

        lax.fori_loop(0, (n_chunks - 3) // 2, body, 0)
        c = n_chunks - 3
        half(0, c, True, True)
        half(1, c + 1, False, True)
        half(0, c + 2, False, False)
        plsc.subcore_barrier()
        _copy_out(acc, out_hbm, cid, sid, N)

    return prop


def _row_specs(N, shapes):
    specs = []
    for shp in shapes:
        if len(shp) == 3:
            specs.append(pl.BlockSpec((NC, RB, shp[2]), lambda i: (0, i, 0)))
        elif shp[0] == N:
            specs.append(pl.BlockSpec((RB, shp[1]), lambda i: (i, 0)))
        else:
            specs.append(pl.BlockSpec(shp, lambda i: (0,) * len(shp)))
    return specs


def _prep(cnt, x, W1):
    N, F = x.shape
    H = W1.shape[1]

    def body(cnt_ref, x_ref, w_ref, dinvt_ref, xs_ref):
        c = cnt_ref[0, :, 0:1] + cnt_ref[1, :, 0:1]
        dinv = lax.rsqrt(c + 1.0)
        dinvt_ref[...] = jnp.broadcast_to(dinv, (RB, 16))
        xs_ref[...] = dinv * jnp.dot(x_ref[...], w_ref[...],
                                     preferred_element_type=jnp.float32)

    return pl.pallas_call(
        body,
        grid=(N // RB,),
        in_specs=_row_specs(N, [cnt.shape, x.shape, W1.shape]),
        out_specs=_row_specs(N, [(N, 16), (N, H)]),
        out_shape=(jax.ShapeDtypeStruct((N, 16), jnp.float32),
                   jax.ShapeDtypeStruct((N, H), jnp.float32)),
    )(cnt, x, W1)


def _weights(W4, W2, b4):
    F, H2 = W4.shape
    C = W2.shape[1]

    def body(w4_ref, w2_ref, b4_ref, w42_ref, v_ref):
        zw = jnp.zeros((F, F - C), jnp.float32)
        zv = jnp.zeros((1, F - C), jnp.float32)
        w42_ref[...] = jnp.concatenate(
            [jnp.dot(w4_ref[...], w2_ref[...],
                     preferred_element_type=jnp.float32), zw], axis=1)
        v_ref[...] = jnp.concatenate(
            [jnp.dot(b4_ref[...], w2_ref[...],
                     preferred_element_type=jnp.float32), zv], axis=1)

    return pl.pallas_call(
        body,
        out_shape=(jax.ShapeDtypeStruct((F, F), jnp.float32),
                   jax.ShapeDtypeStruct((1, F), jnp.float32)),
    )(W4, W2, b4.reshape(1, -1))


def _layer(parts, xs, dinvt, b, W):
    N, D = xs.shape
    Dout = W.shape[1]

    def body(p_ref, xs_ref, dt_ref, b_ref, w_ref, o_ref):
        dinv = dt_ref[:, 0:1]
        h = dinv * (p_ref[0] + p_ref[1] + xs_ref[...]) + b_ref[...]
        h = jnp.tanh(h)
        o_ref[...] = dinv * jnp.dot(h, w_ref[...],
                                    preferred_element_type=jnp.float32)

    return pl.pallas_call(
        body,
        grid=(N // RB,),
        in_specs=_row_specs(N, [parts.shape, xs.shape, dinvt.shape,
                                (1, D), W.shape]),
        out_specs=_row_specs(N, [(N, Dout)])[0],
        out_shape=jax.ShapeDtypeStruct((N, Dout), jnp.float32),
    )(parts, xs, dinvt, b.reshape(1, -1), W)


def _rescale(parts, xs, dinvt, v):
    N, D = xs.shape

    def body(p_ref, xs_ref, dt_ref, v_ref, o_ref):
        dinv = dt_ref[:, 0:1]
        o_ref[...] = dinv * (dinv * (p_ref[0] + p_ref[1] + xs_ref[...])
                             + v_ref[...])

    return pl.pallas_call(
        body,
        grid=(N // RB,),
        in_specs=_row_specs(N, [parts.shape, xs.shape, dinvt.shape, (1, D)]),
        out_specs=_row_specs(N, [(N, D)])[0],
        out_shape=jax.ShapeDtypeStruct((N, D), jnp.float32),
    )(parts, xs, dinvt, v)


def _final(parts, xs, dinvt, b2, C):
    N, D = xs.shape

    def body(p_ref, xs_ref, dt_ref, b2_ref, o_ref):
        dinv = dt_ref[:, 0:1]
        u = dinv * (p_ref[0] + p_ref[1] + xs_ref[...])
        o_ref[...] = u[:, :C] + b2_ref[...]

    return pl.pallas_call(
        body,
        grid=(N // RB,),
        in_specs=_row_specs(N, [parts.shape, xs.shape, dinvt.shape, (1, C)]),
        out_specs=_row_specs(N, [(N, C)])[0],
        out_shape=jax.ShapeDtypeStruct((N, C), jnp.float32),
    )(parts, xs, dinvt, b2.reshape(1, -1))


def kernel(x, adj, W1, b1, W3, b3, W4, b4, W2, b2):
    N, F = x.shape
    E = adj.shape[1]
    H = W1.shape[1]
    C = W2.shape[1]
    e_pt = E // (NC * NS)
    row = adj[0].astype(jnp.int32).reshape(NC * NS, e_pt // K, 1, K)
    col = adj[1].astype(jnp.int32).reshape(NC * NS, e_pt // K, 1, K)

    prop_h = _make_prop(N, E, H)

    cnt = _make_count(N, E)(col)
    dinvt, xs = _prep(cnt, x, W1)
    W42, v = _weights(W4, W2, b4)

    p = prop_h(xs, row, col)
    xs = _layer(p, xs, dinvt, b1, W3)
    p = prop_h(xs, row, col)
    xs = _layer(p, xs, dinvt, b3, W3)
    p = prop_h(xs, row, col)
    xs = _layer(p, xs, dinvt, b3, W42)
    p = prop_h(xs, row, col)
    xs = _rescale(p, xs, dinvt, v)
    p = prop_h(xs, row, col)
    return _final(p, xs, dinvt, b2, C)
